# Initial kernel scaffold; baseline (speedup 1.0000x reference)
import jax
import jax.numpy as jnp
from jax.experimental import pallas as pl


def kernel(x, edge_attr, edge_index, predict_edge_index, c1_Wl, c1_bl, c1_Wr, c2_Wl, c2_bl, c2_Wr, pm_W1, pm_b1, pm_W2, pm_b2, ep_W1, ep_b1, ep_W2, ep_b2):
    raise NotImplementedError("write your pallas kernel here")


# speedup vs baseline: 4.4232x; 4.4232x over previous
import functools

import jax
import jax.numpy as jnp
from jax import lax
from jax.experimental import pallas as pl
from jax.experimental.pallas import tpu as pltpu
from jax.experimental.pallas import tpu_sc as plsc

N = 10000
E = 320000
D = 128

NC = 2
NS = 16
NW = NC * NS
L = 16

EPT = E // NW
AGG_B = 100
AGG_NB = EPT // AGG_B
EDG_B = 80
EDG_NB = EPT // EDG_B
N_PAD = 10240
STRIPE = N_PAD // NS
DEG_PAD = 10240
DEG_STRIPE = DEG_PAD // NS

_f32 = jnp.float32


def _dotT(a, w):
    return lax.dot_general(a, w, (((1,), (1,)), ((), ())),
                           preferred_element_type=_f32)


_R = 1000
_GRID = N // _R


def _row_spec():
    return pl.BlockSpec((_R, D), lambda i: (i, 0))


def _w_spec():
    return pl.BlockSpec((D, D), lambda i: (0, 0))


def _b_spec():
    return pl.BlockSpec((1, D), lambda i: (0, 0))


def _col_spec():
    return pl.BlockSpec((_R, 1), lambda i: (i, 0))


def _stage_a_body(x_ref, wl_ref, wr_ref, bl_ref, a_ref, r_ref):
    x = x_ref[...]
    a_ref[...] = _dotT(x, wl_ref[...])
    r_ref[...] = _dotT(x, wr_ref[...]) + bl_ref[...]


def _stage_a(x, wl, wr, bl):
    return pl.pallas_call(
        _stage_a_body,
        grid=(_GRID,),
        in_specs=[_row_spec(), _w_spec(), _w_spec(), _b_spec()],
        out_specs=[_row_spec(), _row_spec()],
        out_shape=[jax.ShapeDtypeStruct((N, D), _f32)] * 2,
    )(x, wl, wr, bl.reshape(1, D))


def _stage_b_body(sa_ref, sb_ref, inv_ref, r_ref, wl_ref, wr_ref, bl_ref,
                  a_ref, r2_ref):
    h = jnp.maximum((sa_ref[...] + sb_ref[...]) * inv_ref[...] + r_ref[...],
                    0.0)
    a_ref[...] = _dotT(h, wl_ref[...])
    r2_ref[...] = _dotT(h, wr_ref[...]) + bl_ref[...]


def _stage_b(sa, sb, invd, r, wl, wr, bl):
    return pl.pallas_call(
        _stage_b_body,
        grid=(_GRID,),
        in_specs=[_row_spec(), _row_spec(), _col_spec(), _row_spec(),
                  _w_spec(), _w_spec(), _b_spec()],
        out_specs=[_row_spec(), _row_spec()],
        out_shape=[jax.ShapeDtypeStruct((N, D), _f32)] * 2,
    )(sa, sb, invd, r, wl, wr, bl.reshape(1, D))


def _stage_c_body(sa_ref, sb_ref, inv_ref, r_ref, w1_ref, b1_ref, w2_ref,
                  b2_ref, wa_ref, wb_ref, eb1_ref, p_ref, q_ref):
    h2 = jnp.maximum((sa_ref[...] + sb_ref[...]) * inv_ref[...] + r_ref[...],
                     0.0)
    t = jnp.maximum(_dotT(h2, w1_ref[...]) + b1_ref[...], 0.0)
    h = _dotT(t, w2_ref[...]) + b2_ref[...]
    p_ref[...] = _dotT(h, wa_ref[...]) + eb1_ref[...]
    q_ref[...] = _dotT(h, wb_ref[...])


def _stage_c(sa, sb, invd, r, pm_w1, pm_b1, pm_w2, pm_b2, w1a, w1b, ep_b1):
    return pl.pallas_call(
        _stage_c_body,
        grid=(_GRID,),
        in_specs=[_row_spec(), _row_spec(), _col_spec(), _row_spec(),
                  _w_spec(), _b_spec(), _w_spec(), _b_spec(),
                  _w_spec(), _w_spec(), _b_spec()],
        out_specs=[_row_spec(), _row_spec()],
        out_shape=[jax.ShapeDtypeStruct((N, D), _f32)] * 2,
    )(sa, sb, invd, r, pm_w1, pm_b1.reshape(1, D), pm_w2,
      pm_b2.reshape(1, D), w1a, w1b, ep_b1.reshape(1, D))


_SC_MESH = dict(core_axis_name="c", subcore_axis_name="s")


def _agg_deg_kernel(a_hbm, src_hbm, dst_hbm, znd_hbm, zdeg_hbm,
                    out_hbm, deg_hbm,
                    sidx, didx, buf, ones, table, degtab, sem):
    c = lax.axis_index("c")
    s = lax.axis_index("s")
    wid = s * NC + c
    pltpu.sync_copy(src_hbm.at[wid], sidx)
    pltpu.sync_copy(dst_hbm.at[wid], didx)
    pltpu.sync_copy(znd_hbm.at[pl.ds(s * STRIPE, STRIPE)],
                    table.at[pl.ds(s * STRIPE, STRIPE)])
    pltpu.sync_copy(zdeg_hbm.at[pl.ds(s * DEG_STRIPE, DEG_STRIPE)],
                    degtab.at[pl.ds(s * DEG_STRIPE, DEG_STRIPE)])
    one = jnp.full((L,), 1.0, _f32)
    for off in (0, 16, 32, 48, 64, 80, 84):
        ones[pl.ds(off, L)] = one
    plsc.subcore_barrier()

    def body(j, carry):
        pltpu.async_copy(a_hbm.at[sidx.at[j]], buf, sem).wait()
        pltpu.sync_copy(buf, table.at[didx.at[j]], add=True)
        pltpu.sync_copy(ones, degtab.at[didx.at[j]], add=True)
        return carry

    lax.fori_loop(0, AGG_NB, body, 0)
    plsc.subcore_barrier()
    pltpu.sync_copy(table.at[pl.ds(s * STRIPE, STRIPE)],
                    out_hbm.at[c, pl.ds(s * STRIPE, STRIPE)])
    pltpu.sync_copy(degtab.at[pl.ds(s * DEG_STRIPE, DEG_STRIPE)],
                    deg_hbm.at[c, pl.ds(s * DEG_STRIPE, DEG_STRIPE)])


def _agg_kernel(a_hbm, src_hbm, dst_hbm, znd_hbm,
                out_hbm,
                sidx, didx, buf, table, sem):
    c = lax.axis_index("c")
    s = lax.axis_index("s")
    wid = s * NC + c
    pltpu.sync_copy(src_hbm.at[wid], sidx)
    pltpu.sync_copy(dst_hbm.at[wid], didx)
    pltpu.sync_copy(znd_hbm.at[pl.ds(s * STRIPE, STRIPE)],
                    table.at[pl.ds(s * STRIPE, STRIPE)])
    plsc.subcore_barrier()

    def body(j, carry):
        pltpu.async_copy(a_hbm.at[sidx.at[j]], buf, sem).wait()
        pltpu.sync_copy(buf, table.at[didx.at[j]], add=True)
        return carry

    lax.fori_loop(0, AGG_NB, body, 0)
    plsc.subcore_barrier()
    pltpu.sync_copy(table.at[pl.ds(s * STRIPE, STRIPE)],
                    out_hbm.at[c, pl.ds(s * STRIPE, STRIPE)])


def _segment_mean_parts(a, src3, dst3, znd, zdeg, with_deg):
    if with_deg:
        out_type = [jax.ShapeDtypeStruct((NC, N_PAD, D), _f32),
                    jax.ShapeDtypeStruct((NC, DEG_PAD), _f32)]
        scratch = [pltpu.VMEM((AGG_NB, AGG_B), jnp.int32),
                   pltpu.VMEM((AGG_NB, AGG_B), jnp.int32),
                   pltpu.VMEM((AGG_B, D), _f32),
                   pltpu.VMEM((AGG_B,), _f32),
                   pltpu.VMEM_SHARED((N_PAD, D), _f32),
                   pltpu.VMEM_SHARED((DEG_PAD,), _f32),
                   pltpu.SemaphoreType.DMA]
        fn = pl.kernel(_agg_deg_kernel, out_type=out_type,
                       mesh=plsc.VectorSubcoreMesh(**_SC_MESH),
                       scratch_types=scratch)
        return fn(a, src3, dst3, znd, zdeg)
    out_type = [jax.ShapeDtypeStruct((NC, N_PAD, D), _f32)]
    scratch = [pltpu.VMEM((AGG_NB, AGG_B), jnp.int32),
               pltpu.VMEM((AGG_NB, AGG_B), jnp.int32),
               pltpu.VMEM((AGG_B, D), _f32),
               pltpu.VMEM_SHARED((N_PAD, D), _f32),
               pltpu.SemaphoreType.DMA]
    fn = pl.kernel(_agg_kernel, out_type=out_type,
                   mesh=plsc.VectorSubcoreMesh(**_SC_MESH),
                   scratch_types=scratch)
    return fn(a, src3, dst3, znd)


def _edge_kernel(p_hbm, q_hbm, pi_hbm, qi_hbm, w2_hbm,
                 y_hbm,
                 pidx, qidx, bufp, bufq, scr, w2v, sem):
    c = lax.axis_index("c")
    s = lax.axis_index("s")
    wid = s * NC + c
    pltpu.sync_copy(pi_hbm.at[wid], pidx)
    pltpu.sync_copy(qi_hbm.at[wid], qidx)
    pltpu.sync_copy(w2_hbm, w2v)
    zero = jnp.zeros((L,), _f32)

    def batch(j, carry):
        pltpu.async_copy(p_hbm.at[pidx.at[j]], bufp, sem).wait()
        pltpu.async_copy(q_hbm.at[qidx.at[j]], bufq, sem).wait()

        def edge(e, cc):
            acc = zero
            for ch in range(D // L):
                sl = pl.ds(ch * L, L)
                t = jnp.maximum(bufp[e, sl] + bufq[e, sl], 0.0)
                acc = acc + t * w2v[sl]
            scr[e] = acc
            return cc

        lax.fori_loop(0, EDG_B, edge, 0)
        pltpu.sync_copy(scr, y_hbm.at[wid, j])
        return carry

    lax.fori_loop(0, EDG_NB, batch, 0)


def _edge_predict(p, q, pi3, qi3, w2):
    scratch = [pltpu.VMEM((EDG_NB, EDG_B), jnp.int32),
               pltpu.VMEM((EDG_NB, EDG_B), jnp.int32),
               pltpu.VMEM((EDG_B, D), _f32),
               pltpu.VMEM((EDG_B, D), _f32),
               pltpu.VMEM((EDG_B, L), _f32),
               pltpu.VMEM((D,), _f32),
               pltpu.SemaphoreType.DMA]
    fn = pl.kernel(_edge_kernel,
                   out_type=jax.ShapeDtypeStruct((NW, EDG_NB, EDG_B, L), _f32),
                   mesh=plsc.VectorSubcoreMesh(**_SC_MESH),
                   scratch_types=scratch)
    return fn(p, q, pi3, qi3, w2)


_ZR = 4000
_ZROWS = E * L // D


def _fold_body(z_ref, s_ref, y_ref):
    y_ref[...] = lax.dot_general(z_ref[...], s_ref[...],
                                 (((1,), (0,)), ((), ())),
                                 preferred_element_type=_f32)


def _lane_fold(z, sel):
    return pl.pallas_call(
        _fold_body,
        grid=(_ZROWS // _ZR,),
        in_specs=[pl.BlockSpec((_ZR, D), lambda i: (i, 0)),
                  pl.BlockSpec((D, 8), lambda i: (0, 0))],
        out_specs=pl.BlockSpec((_ZR, 8), lambda i: (i, 0)),
        out_shape=jax.ShapeDtypeStruct((_ZROWS, 8), _f32),
    )(z, sel)


def kernel(x, edge_attr, edge_index, predict_edge_index,
           c1_Wl, c1_bl, c1_Wr, c2_Wl, c2_bl, c2_Wr,
           pm_W1, pm_b1, pm_W2, pm_b2,
           ep_W1, ep_b1, ep_W2, ep_b2):
    del edge_attr
    src3 = edge_index[0].reshape(NW, AGG_NB, AGG_B)
    dst3 = edge_index[1].reshape(NW, AGG_NB, AGG_B)
    pi3 = predict_edge_index[0].reshape(NW, EDG_NB, EDG_B)
    qi3 = predict_edge_index[1].reshape(NW, EDG_NB, EDG_B)
    znd = jnp.zeros((N_PAD, D), _f32)
    zdeg = jnp.zeros((DEG_PAD,), _f32)

    a1, r1 = _stage_a(x, c1_Wl, c1_Wr, c1_bl)
    s1p, degp = _segment_mean_parts(a1, src3, dst3, znd, zdeg, True)
    deg = (degp[0] + degp[1])[:N]
    invd = (1.0 / jnp.clip(deg, 1.0, None)).reshape(N, 1)

    a2, r2 = _stage_b(s1p[0, :N], s1p[1, :N], invd, r1, c2_Wl, c2_Wr, c2_bl)
    s2p = _segment_mean_parts(a2, src3, dst3, znd, None, False)[0]

    w1a = ep_W1[:, :D]
    w1b = ep_W1[:, D:]
    p, q = _stage_c(s2p[0, :N], s2p[1, :N], invd, r2, pm_W1, pm_b1, pm_W2, pm_b2,
                    w1a, w1b, ep_b1)

    y16 = _edge_predict(p, q, pi3, qi3, ep_W2.reshape(D))
    sel = (jnp.arange(D)[:, None] // L == jnp.arange(8)[None, :]).astype(_f32)
    y = _lane_fold(y16.reshape(_ZROWS, D), sel)
    return y.reshape(E, 1) + ep_b2
